# baseline (speedup 1.0000x reference)
import jax
import jax.numpy as jnp
from jax import lax
from jax.experimental import pallas as pl
from jax.experimental.pallas import tpu as pltpu
from jax.experimental.pallas import tpu_sc as plsc

NC = 2
NS = 16
NW = NC * NS

V = 1000000
B = 16384 * 26
D = 64

CHUNK = 416
ROWS_PER_CHUNK = CHUNK // 26
B_PER_W = B // NW
N_CHUNKS = B_PER_W // CHUNK


def _gather_body(idx_hbm, wide_hbm, out_hbm, idx_v, rows_v, out_v, sem):
    wid = lax.axis_index("s") * NC + lax.axis_index("c")
    wbase = wid * B_PER_W

    def chunk(i, carry):
        base = wbase + i * CHUNK
        pltpu.sync_copy(idx_hbm.at[pl.ds(base, CHUNK)], idx_v)
        pltpu.async_copy(wide_hbm.at[idx_v], rows_v, sem).wait()

        def compact(j, carry2):
            for k in range(D // 16):
                out_v[j, pl.ds(k * 16, 16)] = rows_v[j, pl.ds(k * 16, 16)]
            return carry2

        lax.fori_loop(0, CHUNK, compact, 0)
        row0 = base // 26

        def writeback(j, carry3):
            pltpu.sync_copy(
                out_v.at[pl.ds(j * 26, 26)], out_hbm.at[row0 + j]
            )
            return carry3

        lax.fori_loop(0, ROWS_PER_CHUNK, writeback, 0)
        return carry

    lax.fori_loop(0, N_CHUNKS, chunk, 0)


def kernel(indices, embedding_matrix):
    idx_flat = indices.reshape(-1).astype(jnp.int32)
    wide = jnp.pad(embedding_matrix, ((0, 0), (0, D)))
    mesh = plsc.VectorSubcoreMesh(core_axis_name="c", subcore_axis_name="s")
    gather = pl.kernel(
        _gather_body,
        out_type=jax.ShapeDtypeStruct(
            (indices.shape[0], indices.shape[1], D), jnp.float32
        ),
        mesh=mesh,
        scratch_types=[
            pltpu.VMEM((CHUNK,), jnp.int32),
            pltpu.VMEM((CHUNK, 2 * D), jnp.float32),
            pltpu.VMEM((CHUNK, D), jnp.float32),
            pltpu.SemaphoreType.DMA,
        ],
        compiler_params=pltpu.CompilerParams(use_tc_tiling_on_sc=True),
    )
    return gather(idx_flat, wide)
